# baseline (speedup 1.0000x reference)
import functools
import jax
import jax.numpy as jnp
from jax import lax
from jax.experimental import pallas as pl
from jax.experimental.pallas import tpu as pltpu
from jax.experimental.pallas import tpu_sc as plsc

PAD = 1
L = 16
CH = 32


def _make_sc(bsz, seq_len, d):
    info = plsc.get_sparse_core_info()
    nc = info.num_cores
    nw = nc * info.num_subcores
    js_w = seq_len // nw
    nch = js_w // CH
    assert seq_len % nw == 0 and js_w % CH == 0
    mesh = plsc.VectorSubcoreMesh(core_axis_name="c", subcore_axis_name="s")

    @functools.partial(
        pl.kernel,
        mesh=mesh,
        out_type=jax.ShapeDtypeStruct((bsz * seq_len, d), jnp.float32),
        scratch_types=[
            pltpu.VMEM((bsz, js_w), jnp.int32),
            pltpu.VMEM((CH, d), jnp.float32),
            pltpu.VMEM((CH, d), jnp.float32),
            pltpu.VMEM((CH, d), jnp.float32),
            pltpu.VMEM((nch, CH), jnp.int32),
            pltpu.VMEM((CH,), jnp.int32),
            pltpu.SemaphoreType.DMA,
            pltpu.SemaphoreType.DMA,
            pltpu.SemaphoreType.DMA,
            pltpu.SemaphoreType.DMA,
            pltpu.SemaphoreType.DMA,
            pltpu.SemaphoreType.DMA,
        ],
    )
    def k(inp_hbm, table_hbm, out_hbm, tok_v, buf0, buf1, buf2, iidx, fidx,
          g0, g1, g2, s0, s1, s2):
        wid = lax.axis_index("c") * (nw // nc) + lax.axis_index("s")
        j0 = pl.multiple_of(wid * js_w, js_w)

        for b in range(bsz):
            pltpu.sync_copy(
                inp_hbm.at[pl.ds(b * seq_len + j0, js_w)], tok_v.at[b]
            )

        bufs = (buf0, buf1, buf2)
        gsems = (g0, g1, g2)
        ssems = (s0, s1, s2)
        nbuf = len(bufs)
        lane = jnp.arange(L, dtype=jnp.int32)

        for ch in range(nch):
            for v in range(CH // L):
                iidx[ch, pl.ds(v * L, L)] = lane + (j0 + ch * CH + v * L + 2)

        has_pad = []
        for b in range(bsz):
            acc = jnp.where(tok_v[b, pl.ds(0, L)] == PAD, 1, 0)
            for v in range(1, js_w // L):
                tok = tok_v[b, pl.ds(v * L, L)]
                acc = acc | jnp.where(tok == PAD, 1, 0)
            s = acc[0]
            for i in range(1, L):
                s = s | acc[i]
            has_pad.append(s > 0)

        def clean_gather(ch, p):
            return pltpu.make_async_copy(
                table_hbm.at[iidx.at[ch]], bufs[p], gsems[p]
            )

        def out_slice(b, ch):
            start = pl.multiple_of(b * seq_len + j0 + ch * CH, 8)
            return out_hbm.at[pl.ds(start, CH)]

        clean_gather(0, 0).start()
        for ch in range(nch):
            p = ch % nbuf
            clean_gather(ch, p).wait()
            if ch + 1 < nch:
                q = (ch + 1) % nbuf
                if ch >= nbuf - 1:
                    for b in range(bsz):
                        pltpu.make_async_copy(
                            bufs[q], out_slice(b, ch + 1 - nbuf), ssems[q]
                        ).wait()
                clean_gather(ch + 1, q).start()
            for b in range(bsz):
                pltpu.make_async_copy(bufs[p], out_slice(b, ch), ssems[p]).start()
        for ch in range(nch - nbuf, nch):
            p = ch % nbuf
            for b in range(bsz):
                pltpu.make_async_copy(bufs[p], out_slice(b, ch), ssems[p]).wait()

        for b in range(bsz):

            @pl.when(has_pad[b])
            def _fixup(b=b):
                for ch in range(nch):
                    for v in range(CH // L):
                        tok = tok_v[b, pl.ds(ch * CH + v * L, L)]
                        pos = lane + (j0 + ch * CH + v * L + 2)
                        fidx[pl.ds(v * L, L)] = jnp.where(tok != PAD, pos, PAD)
                    pltpu.make_async_copy(table_hbm.at[fidx], buf0, g0).start()
                    pltpu.make_async_copy(table_hbm.at[fidx], buf0, g0).wait()
                    pltpu.make_async_copy(buf0, out_slice(b, ch), s0).start()
                    pltpu.make_async_copy(buf0, out_slice(b, ch), s0).wait()

    return k


def kernel(input, weights):
    bsz, seq_len = input.shape
    d = weights.shape[1]
    k = _make_sc(bsz, seq_len, d)
    out = k(input.reshape(-1), weights)
    return out.reshape(bsz, seq_len, d)
